# baseline (speedup 1.0000x reference)
import jax
import jax.numpy as jnp
from jax.experimental import pallas as pl
from jax.experimental.pallas import tpu as pltpu

B, N, D = 16, 4096, 256
NCHUNK = 4
CHUNK = N // NCHUNK


def _body(x_ref, m_ref, o_ref, cnt_ref):
    j = pl.program_id(1)

    m = m_ref[0, 0, :]
    x = x_ref[0]
    s = jnp.dot(m, x, preferred_element_type=jnp.float32)
    c = jnp.sum(m)

    @pl.when(j == 0)
    def _init():
        o_ref[0, 0, :] = s
        cnt_ref[0, 0] = c

    @pl.when(j > 0)
    def _acc():
        o_ref[0, 0, :] += s
        cnt_ref[0, 0] += c

    @pl.when(j == NCHUNK - 1)
    def _fin():
        o_ref[0, 0, :] = o_ref[0, 0, :] / cnt_ref[0, 0]


def kernel(inputs, mask):
    m3 = mask.astype(jnp.float32).reshape(B, 1, N)
    out = pl.pallas_call(
        _body,
        grid=(B, NCHUNK),
        in_specs=[
            pl.BlockSpec((1, CHUNK, D), lambda b, j: (b, j, 0)),
            pl.BlockSpec((1, 1, CHUNK), lambda b, j: (b, 0, j)),
        ],
        out_specs=pl.BlockSpec((1, 1, D), lambda b, j: (b, 0, 0)),
        out_shape=jax.ShapeDtypeStruct((B, 1, D), jnp.float32),
        scratch_shapes=[pltpu.SMEM((1, 1), jnp.float32)],
    )(inputs, m3)
    return out.reshape(B, D)
